# baseline (speedup 1.0000x reference)
import functools

import jax
import jax.numpy as jnp
from jax import lax
from jax.experimental import pallas as pl
from jax.experimental.pallas import tpu as pltpu
from jax.experimental.pallas import tpu_sc as plsc
from jax.experimental import layout as jex_layout

D = 64
NNEG = 4
NITEM = NNEG + 1


def _sc_worker_count():
    try:
        info = plsc.get_sparse_core_info()
        return info.num_cores, info.num_subcores
    except Exception:
        return 2, 16


def _pair_body(at_ref, bt_ref, out_ref):
    eye = jnp.eye(D, dtype=jnp.float32)
    dn = (((0,), (0,)), ((), ()))
    a = jax.lax.dot_general(at_ref[...], eye, dn,
                            preferred_element_type=jnp.float32)
    b = jax.lax.dot_general(bt_ref[...], eye, dn,
                            preferred_element_type=jnp.float32)
    out_ref[...] = jnp.concatenate([a, b], axis=1)


def _pair_concat_t(a_t, b_t, cols_per_block):
    n = a_t.shape[1]
    grid = (pl.cdiv(n, cols_per_block),)
    spec = pl.BlockSpec((D, cols_per_block), lambda i: (0, i))
    return pl.pallas_call(
        _pair_body,
        grid=grid,
        in_specs=[spec, spec],
        out_specs=pl.BlockSpec((cols_per_block, 2 * D), lambda i: (i, 0)),
        out_shape=jax.ShapeDtypeStruct((n, 2 * D), jnp.float32),
    )(a_t, b_t)


@functools.lru_cache(maxsize=None)
def _make_gather(B, nc, ns):
    nw = nc * ns
    bpw = B // nw
    ipw = NITEM * bpw
    nchunk = NITEM
    mesh = plsc.VectorSubcoreMesh(core_axis_name="c", subcore_axis_name="s")

    @functools.partial(
        pl.kernel,
        mesh=mesh,
        out_type=[
            jax.ShapeDtypeStruct((B, 2 * D), jnp.float32),
            jax.ShapeDtypeStruct((NITEM * B, 2 * D), jnp.float32),
        ],
        scratch_types=[
            pltpu.VMEM((bpw,), jnp.int32),
            pltpu.VMEM((ipw,), jnp.int32),
            pltpu.VMEM((bpw, 2 * D), jnp.float32),
            pltpu.VMEM((ipw, 2 * D), jnp.float32),
            pltpu.SemaphoreType.DMA,
        ],
    )
    def gk(user1d, items1d, u_table, i_table,
           out_u, out_i,
           idx_u, idx_it, r_u, r_it, sem):
        wid = lax.axis_index("s") * nc + lax.axis_index("c")
        pltpu.sync_copy(user1d.at[pl.ds(wid * bpw, bpw)], idx_u)
        pltpu.sync_copy(items1d.at[pl.ds(wid * ipw, ipw)], idx_it)
        cps = [pltpu.async_copy(u_table.at[idx_u], r_u, sem)]
        for j in range(nchunk):
            src = idx_it.at[pl.ds(j * bpw, bpw)]
            dst = pl.ds(j * bpw, bpw)
            cps.append(pltpu.async_copy(i_table.at[src], r_it.at[dst], sem))
        for c in cps:
            c.wait()
        pltpu.sync_copy(r_u, out_u.at[pl.ds(wid * bpw, bpw)])
        pltpu.sync_copy(r_it, out_i.at[pl.ds(wid * ipw, ipw)])

    return gk


def _dense_body(u_ref, it_ref,
                w1_ref, b1_ref, w2_ref, b2_ref, w3_ref, b3_ref,
                w4_ref, b4_ref, wd_ref, bd_ref, out_ref):
    r = u_ref.shape[0]
    u = u_ref[...]
    mfu = u[:, :D]
    mlu = u[:, D:]
    sig_parts = []
    x_parts = []
    for k in range(NITEM):
        it = it_ref[k]
        sig_parts.append(jax.nn.sigmoid(mfu * it[:, :D]))
        x_parts.append(jnp.concatenate([mlu, it[:, D:]], axis=1))
    sig = jnp.concatenate(sig_parts, axis=0)
    x = jnp.concatenate(x_parts, axis=0)
    for w_ref, b_ref in ((w1_ref, b1_ref), (w2_ref, b2_ref),
                         (w3_ref, b3_ref), (w4_ref, b4_ref)):
        x = jnp.maximum(
            jnp.dot(x, w_ref[...], preferred_element_type=jnp.float32)
            + b_ref[...], 0.0)
    feat = jnp.concatenate([sig, x], axis=1)
    scores = jnp.dot(feat, wd_ref[...], preferred_element_type=jnp.float32) \
        + bd_ref[...]
    s = [scores[k * r:(k + 1) * r] for k in range(NITEM)]
    out_ref[...] = jnp.concatenate(
        [s[0], s[0], s[0], s[0], s[1], s[2], s[3], s[4]], axis=1)


def _dense(u_rows, it_rows3, W1, b1, W2, b2, W3, b3, W4, b4, Wd, bd):
    B = u_rows.shape[0]
    R = 512
    grid = (B // R,)
    full = lambda shape: pl.BlockSpec(shape, lambda i: tuple(0 for _ in shape))
    in_specs = [
        pl.BlockSpec((R, 2 * D), lambda i: (i, 0)),
        pl.BlockSpec((NITEM, R, 2 * D), lambda i: (0, i, 0)),
        full(W1.shape), full((1, b1.shape[0])),
        full(W2.shape), full((1, b2.shape[0])),
        full(W3.shape), full((1, b3.shape[0])),
        full(W4.shape), full((1, b4.shape[0])),
        full(Wd.shape), full((1, 1)),
    ]
    return pl.pallas_call(
        _dense_body,
        grid=grid,
        in_specs=in_specs,
        out_specs=pl.BlockSpec((R, 2 * NNEG), lambda i: (i, 0)),
        out_shape=jax.ShapeDtypeStruct((B, 2 * NNEG), jnp.float32),
    )(u_rows, it_rows3,
      W1, b1.reshape(1, -1), W2, b2.reshape(1, -1),
      W3, b3.reshape(1, -1), W4, b4.reshape(1, -1),
      Wd, bd.reshape(1, 1))


def kernel(user, pos_item, neg_item, mf_user_table, mf_item_table,
           mlp_user_table, mlp_item_table,
           W1, b1, W2, b2, W3, b3, W4, b4, Wd, bd):
    B = user.shape[0]
    nc, ns = _sc_worker_count()
    nw = nc * ns
    user1d = user.astype(jnp.int32)
    items = jnp.concatenate(
        [pos_item.astype(jnp.int32)[None, :], neg_item.astype(jnp.int32).T],
        axis=0)
    items1d = items.reshape(NITEM * B)
    u_comb = _pair_concat_t(mf_user_table.T, mlp_user_table.T, 1024)
    i_comb = _pair_concat_t(mf_item_table.T, mlp_item_table.T, 1024)
    gk = _make_gather(B, nc, ns)
    u_rows, it_rows = gk(user1d, items1d, u_comb, i_comb)
    it_rows3 = it_rows.reshape(NITEM, B, 2 * D)
    return _dense(u_rows, it_rows3,
                  W1, b1, W2, b2, W3, b3, W4, b4, Wd, bd)


# speedup vs baseline: 1.5455x; 1.5455x over previous
import functools

import jax
import jax.numpy as jnp
from jax import lax
from jax.experimental import pallas as pl
from jax.experimental.pallas import tpu as pltpu
from jax.experimental.pallas import tpu_sc as plsc
from jax.experimental import layout as jex_layout

D = 64
NNEG = 4
NITEM = NNEG + 1


def _sc_worker_count():
    try:
        info = plsc.get_sparse_core_info()
        return info.num_cores, info.num_subcores
    except Exception:
        return 2, 16


def _pair_body(au_ref, bu_ref, ai_ref, bi_ref, outu_ref, outi_ref):
    outu_ref[...] = jnp.concatenate([au_ref[...].T, bu_ref[...].T], axis=1)
    outi_ref[...] = jnp.concatenate([ai_ref[...].T, bi_ref[...].T], axis=1)


def _pair_concat_t(au_t, bu_t, ai_t, bi_t, cols_per_block):
    n = au_t.shape[1]
    grid = (pl.cdiv(n, cols_per_block),)
    spec = pl.BlockSpec((D, cols_per_block), lambda i: (0, i))
    ospec = pl.BlockSpec((cols_per_block, 2 * D), lambda i: (i, 0))
    oshape = jax.ShapeDtypeStruct((n, 2 * D), jnp.float32)
    return pl.pallas_call(
        _pair_body,
        grid=grid,
        in_specs=[spec, spec, spec, spec],
        out_specs=[ospec, ospec],
        out_shape=[oshape, oshape],
    )(au_t, bu_t, ai_t, bi_t)


@functools.lru_cache(maxsize=None)
def _make_gather(B, nc, ns):
    nw = nc * ns
    bpw = B // nw
    ipw = NITEM * bpw
    nchunk = NITEM
    mesh = plsc.VectorSubcoreMesh(core_axis_name="c", subcore_axis_name="s")

    @functools.partial(
        pl.kernel,
        mesh=mesh,
        out_type=[
            jax.ShapeDtypeStruct((B, 2 * D), jnp.float32),
            jax.ShapeDtypeStruct((NITEM * B, 2 * D), jnp.float32),
        ],
        scratch_types=[
            pltpu.VMEM((bpw,), jnp.int32),
            pltpu.VMEM((ipw,), jnp.int32),
            pltpu.VMEM((bpw, 2 * D), jnp.float32),
            pltpu.VMEM((ipw, 2 * D), jnp.float32),
            pltpu.SemaphoreType.DMA,
        ],
    )
    def gk(user1d, items1d, u_table, i_table,
           out_u, out_i,
           idx_u, idx_it, r_u, r_it, sem):
        wid = lax.axis_index("s") * nc + lax.axis_index("c")
        pltpu.sync_copy(user1d.at[pl.ds(wid * bpw, bpw)], idx_u)
        pltpu.sync_copy(items1d.at[pl.ds(wid * ipw, ipw)], idx_it)
        cps = [pltpu.async_copy(u_table.at[idx_u], r_u, sem)]
        for j in range(nchunk):
            src = idx_it.at[pl.ds(j * bpw, bpw)]
            dst = pl.ds(j * bpw, bpw)
            cps.append(pltpu.async_copy(i_table.at[src], r_it.at[dst], sem))
        for c in cps:
            c.wait()
        pltpu.sync_copy(r_u, out_u.at[pl.ds(wid * bpw, bpw)])
        pltpu.sync_copy(r_it, out_i.at[pl.ds(wid * ipw, ipw)])

    return gk


def _dense_body(u_ref, it_ref,
                w1_ref, b1_ref, w2_ref, b2_ref, w3_ref, b3_ref,
                w4_ref, b4_ref, wd_ref, bd_ref, out_ref):
    r = u_ref.shape[0]
    u = u_ref[...]
    mfu = u[:, :D]
    mlu = u[:, D:]
    sig_parts = []
    x_parts = []
    for k in range(NITEM):
        it = it_ref[k]
        sig_parts.append(jax.nn.sigmoid(mfu * it[:, :D]))
        x_parts.append(jnp.concatenate([mlu, it[:, D:]], axis=1))
    sig = jnp.concatenate(sig_parts, axis=0)
    x = jnp.concatenate(x_parts, axis=0)
    for w_ref, b_ref in ((w1_ref, b1_ref), (w2_ref, b2_ref),
                         (w3_ref, b3_ref), (w4_ref, b4_ref)):
        x = jnp.maximum(
            jnp.dot(x, w_ref[...], preferred_element_type=jnp.float32)
            + b_ref[...], 0.0)
    feat = jnp.concatenate([sig, x], axis=1)
    scores = jnp.dot(feat, wd_ref[...], preferred_element_type=jnp.float32) \
        + bd_ref[...]
    s = [scores[k * r:(k + 1) * r] for k in range(NITEM)]
    out_ref[...] = jnp.concatenate(
        [s[0], s[0], s[0], s[0], s[1], s[2], s[3], s[4]], axis=1)


def _dense(u_rows, it_rows3, W1, b1, W2, b2, W3, b3, W4, b4, Wd, bd):
    B = u_rows.shape[0]
    R = 512
    grid = (B // R,)
    full = lambda shape: pl.BlockSpec(shape, lambda i: tuple(0 for _ in shape))
    in_specs = [
        pl.BlockSpec((R, 2 * D), lambda i: (i, 0)),
        pl.BlockSpec((NITEM, R, 2 * D), lambda i: (0, i, 0)),
        full(W1.shape), full((1, b1.shape[0])),
        full(W2.shape), full((1, b2.shape[0])),
        full(W3.shape), full((1, b3.shape[0])),
        full(W4.shape), full((1, b4.shape[0])),
        full(Wd.shape), full((1, 1)),
    ]
    return pl.pallas_call(
        _dense_body,
        grid=grid,
        in_specs=in_specs,
        out_specs=pl.BlockSpec((R, 2 * NNEG), lambda i: (i, 0)),
        out_shape=jax.ShapeDtypeStruct((B, 2 * NNEG), jnp.float32),
    )(u_rows, it_rows3,
      W1, b1.reshape(1, -1), W2, b2.reshape(1, -1),
      W3, b3.reshape(1, -1), W4, b4.reshape(1, -1),
      Wd, bd.reshape(1, 1))


def kernel(user, pos_item, neg_item, mf_user_table, mf_item_table,
           mlp_user_table, mlp_item_table,
           W1, b1, W2, b2, W3, b3, W4, b4, Wd, bd):
    B = user.shape[0]
    nc, ns = _sc_worker_count()
    nw = nc * ns
    user1d = user.astype(jnp.int32)
    items = jnp.concatenate(
        [pos_item.astype(jnp.int32)[None, :], neg_item.astype(jnp.int32).T],
        axis=0)
    items1d = items.reshape(NITEM * B)
    u_comb, i_comb = _pair_concat_t(
        mf_user_table.T, mlp_user_table.T,
        mf_item_table.T, mlp_item_table.T, 2048)
    gk = _make_gather(B, nc, ns)
    u_rows, it_rows = gk(user1d, items1d, u_comb, i_comb)
    it_rows3 = it_rows.reshape(NITEM, B, 2 * D)
    return _dense(u_rows, it_rows3,
                  W1, b1, W2, b2, W3, b3, W4, b4, Wd, bd)


# speedup vs baseline: 1.6744x; 1.0834x over previous
import functools

import jax
import jax.numpy as jnp
from jax import lax
from jax.experimental import pallas as pl
from jax.experimental.pallas import tpu as pltpu
from jax.experimental.pallas import tpu_sc as plsc
from jax.experimental import layout as jex_layout

D = 64
NNEG = 4
NITEM = NNEG + 1


def _sc_worker_count():
    try:
        info = plsc.get_sparse_core_info()
        return info.num_cores, info.num_subcores
    except Exception:
        return 2, 16


def _pair_body(au_ref, bu_ref, ai_ref, bi_ref, outu_ref, outi_ref):
    outu_ref[...] = jnp.concatenate([au_ref[...].T, bu_ref[...].T], axis=1)
    outi_ref[...] = jnp.concatenate([ai_ref[...].T, bi_ref[...].T], axis=1)


def _pair_concat_t(au_t, bu_t, ai_t, bi_t, cols_per_block):
    n = au_t.shape[1]
    grid = (pl.cdiv(n, cols_per_block),)
    spec = pl.BlockSpec((D, cols_per_block), lambda i: (0, i))
    ospec = pl.BlockSpec((cols_per_block, 2 * D), lambda i: (i, 0))
    oshape = jax.ShapeDtypeStruct((n, 2 * D), jnp.float32)
    return pl.pallas_call(
        _pair_body,
        grid=grid,
        in_specs=[spec, spec, spec, spec],
        out_specs=[ospec, ospec],
        out_shape=[oshape, oshape],
    )(au_t, bu_t, ai_t, bi_t)


@functools.lru_cache(maxsize=None)
def _make_gather(B, nc, ns):
    nw = nc * ns
    bpw = B // nw
    ipw = NITEM * bpw
    nchunk = NITEM
    mesh = plsc.VectorSubcoreMesh(core_axis_name="c", subcore_axis_name="s")

    @functools.partial(
        pl.kernel,
        mesh=mesh,
        out_type=[
            jax.ShapeDtypeStruct((B, 2 * D), jnp.float32),
            jax.ShapeDtypeStruct((NITEM * B, 2 * D), jnp.float32),
        ],
        scratch_types=[
            pltpu.VMEM((bpw,), jnp.int32),
            pltpu.VMEM((ipw,), jnp.int32),
            pltpu.VMEM((bpw, 2 * D), jnp.float32),
            pltpu.VMEM((ipw, 2 * D), jnp.float32),
            pltpu.SemaphoreType.DMA,
        ],
    )
    def gk(user1d, items1d, u_table, i_table,
           out_u, out_i,
           idx_u, idx_it, r_u, r_it, sem):
        wid = lax.axis_index("s") * nc + lax.axis_index("c")
        pltpu.sync_copy(user1d.at[pl.ds(wid * bpw, bpw)], idx_u)
        pltpu.sync_copy(items1d.at[pl.ds(wid * ipw, ipw)], idx_it)
        cps = [pltpu.async_copy(u_table.at[idx_u], r_u, sem)]
        for j in range(nchunk):
            src = idx_it.at[pl.ds(j * bpw, bpw)]
            dst = pl.ds(j * bpw, bpw)
            cps.append(pltpu.async_copy(i_table.at[src], r_it.at[dst], sem))
        for c in cps:
            c.wait()
        pltpu.sync_copy(r_u, out_u.at[pl.ds(wid * bpw, bpw)])
        pltpu.sync_copy(r_it, out_i.at[pl.ds(wid * ipw, ipw)])

    return gk


def _dense_body(u_ref, it_ref,
                w1_ref, b1_ref, w2_ref, b2_ref, w3_ref, b3_ref,
                w4_ref, b4_ref, wd_ref, bd_ref, out_ref):
    r = u_ref.shape[0]
    u = u_ref[...]
    mfu = u[:, :D]
    mlu = u[:, D:]
    sig_parts = []
    x_parts = []
    for k in range(NITEM):
        it = it_ref[k]
        sig_parts.append(jax.nn.sigmoid(mfu * it[:, :D]))
        x_parts.append(jnp.concatenate([mlu, it[:, D:]], axis=1))
    sig = jnp.concatenate(sig_parts, axis=0)
    x = jnp.concatenate(x_parts, axis=0)
    for w_ref, b_ref in ((w1_ref, b1_ref), (w2_ref, b2_ref),
                         (w3_ref, b3_ref), (w4_ref, b4_ref)):
        x = jnp.maximum(
            jnp.dot(x, w_ref[...], preferred_element_type=jnp.float32)
            + b_ref[...], 0.0)
    feat = jnp.concatenate([sig, x], axis=1)
    scores = jnp.dot(feat, wd_ref[...], preferred_element_type=jnp.float32) \
        + bd_ref[...]
    s = [scores[k * r:(k + 1) * r] for k in range(NITEM)]
    out_ref[...] = jnp.concatenate(
        [s[0], s[0], s[0], s[0], s[1], s[2], s[3], s[4]], axis=1)


def _dense(u_rows, it_rows3, W1, b1, W2, b2, W3, b3, W4, b4, Wd, bd):
    B = u_rows.shape[0]
    R = 512
    grid = (B // R,)
    full = lambda shape: pl.BlockSpec(shape, lambda i: tuple(0 for _ in shape))
    in_specs = [
        pl.BlockSpec((R, 2 * D), lambda i: (i, 0)),
        pl.BlockSpec((NITEM, R, 2 * D), lambda i: (0, i, 0)),
        full(W1.shape), full((1, b1.shape[0])),
        full(W2.shape), full((1, b2.shape[0])),
        full(W3.shape), full((1, b3.shape[0])),
        full(W4.shape), full((1, b4.shape[0])),
        full(Wd.shape), full((1, 1)),
    ]
    return pl.pallas_call(
        _dense_body,
        grid=grid,
        in_specs=in_specs,
        out_specs=pl.BlockSpec((R, 2 * NNEG), lambda i: (i, 0)),
        out_shape=jax.ShapeDtypeStruct((B, 2 * NNEG), jnp.float32),
    )(u_rows, it_rows3,
      W1, b1.reshape(1, -1), W2, b2.reshape(1, -1),
      W3, b3.reshape(1, -1), W4, b4.reshape(1, -1),
      Wd, bd.reshape(1, 1))


def kernel(user, pos_item, neg_item, mf_user_table, mf_item_table,
           mlp_user_table, mlp_item_table,
           W1, b1, W2, b2, W3, b3, W4, b4, Wd, bd):
    B = user.shape[0]
    nc, ns = _sc_worker_count()
    nw = nc * ns
    user1d = user.astype(jnp.int32)
    items = jnp.concatenate(
        [pos_item.astype(jnp.int32)[None, :], neg_item.astype(jnp.int32).T],
        axis=0)
    items1d = items.reshape(NITEM * B)
    u_comb, i_comb = _pair_concat_t(
        mf_user_table.T, mlp_user_table.T,
        mf_item_table.T, mlp_item_table.T, 4096)
    gk = _make_gather(B, nc, ns)
    u_rows, it_rows = gk(user1d, items1d, u_comb, i_comb)
    it_rows3 = it_rows.reshape(NITEM, B, 2 * D)
    return _dense(u_rows, it_rows3,
                  W1, b1, W2, b2, W3, b3, W4, b4, Wd, bd)


# speedup vs baseline: 1.6943x; 1.0119x over previous
import functools

import jax
import jax.numpy as jnp
from jax import lax
from jax.experimental import pallas as pl
from jax.experimental.pallas import tpu as pltpu
from jax.experimental.pallas import tpu_sc as plsc
from jax.experimental import layout as jex_layout

D = 64
NNEG = 4
NITEM = NNEG + 1


def _sc_worker_count():
    try:
        info = plsc.get_sparse_core_info()
        return info.num_cores, info.num_subcores
    except Exception:
        return 2, 16


def _pair_body(au_ref, bu_ref, ai_ref, bi_ref, outu_ref, outi_ref):
    outu_ref[...] = jnp.concatenate([au_ref[...].T, bu_ref[...].T], axis=1)
    outi_ref[...] = jnp.concatenate([ai_ref[...].T, bi_ref[...].T], axis=1)


def _pair_concat_t(au_t, bu_t, ai_t, bi_t, cols_per_block):
    n = au_t.shape[1]
    grid = (pl.cdiv(n, cols_per_block),)
    spec = pl.BlockSpec((D, cols_per_block), lambda i: (0, i))
    ospec = pl.BlockSpec((cols_per_block, 2 * D), lambda i: (i, 0))
    oshape = jax.ShapeDtypeStruct((n, 2 * D), jnp.float32)
    return pl.pallas_call(
        _pair_body,
        grid=grid,
        in_specs=[spec, spec, spec, spec],
        out_specs=[ospec, ospec],
        out_shape=[oshape, oshape],
    )(au_t, bu_t, ai_t, bi_t)


@functools.lru_cache(maxsize=None)
def _make_gather(B, nc, ns):
    nw = nc * ns
    bpw = B // nw
    ipw = NITEM * bpw
    nchunk = NITEM
    mesh = plsc.VectorSubcoreMesh(core_axis_name="c", subcore_axis_name="s")

    @functools.partial(
        pl.kernel,
        mesh=mesh,
        out_type=[
            jax.ShapeDtypeStruct((B, 2 * D), jnp.float32),
            jax.ShapeDtypeStruct((NITEM * B, 2 * D), jnp.float32),
        ],
        scratch_types=[
            pltpu.VMEM((bpw,), jnp.int32),
            pltpu.VMEM((ipw,), jnp.int32),
            pltpu.VMEM((bpw, 2 * D), jnp.float32),
            pltpu.VMEM((ipw, 2 * D), jnp.float32),
            pltpu.SemaphoreType.DMA,
        ],
    )
    def gk(user1d, items1d, u_table, i_table,
           out_u, out_i,
           idx_u, idx_it, r_u, r_it, sem):
        wid = lax.axis_index("s") * nc + lax.axis_index("c")
        pltpu.sync_copy(user1d.at[pl.ds(wid * bpw, bpw)], idx_u)
        pltpu.sync_copy(items1d.at[pl.ds(wid * ipw, ipw)], idx_it)
        cps = [pltpu.async_copy(u_table.at[idx_u], r_u, sem)]
        for j in range(nchunk):
            src = idx_it.at[pl.ds(j * bpw, bpw)]
            dst = pl.ds(j * bpw, bpw)
            cps.append(pltpu.async_copy(i_table.at[src], r_it.at[dst], sem))
        for c in cps:
            c.wait()
        pltpu.sync_copy(r_u, out_u.at[pl.ds(wid * bpw, bpw)])
        pltpu.sync_copy(r_it, out_i.at[pl.ds(wid * ipw, ipw)])

    return gk


def _dense_body(u_ref, it_ref,
                w1_ref, b1_ref, w2_ref, b2_ref, w3_ref, b3_ref,
                w4_ref, b4_ref, wd_ref, bd_ref, out_ref):
    r = u_ref.shape[0]
    u = u_ref[...]
    mfu = u[:, :D]
    mlu = u[:, D:]
    sig_parts = []
    x_parts = []
    for k in range(NITEM):
        it = it_ref[k]
        sig_parts.append(jax.nn.sigmoid(mfu * it[:, :D]))
        x_parts.append(jnp.concatenate([mlu, it[:, D:]], axis=1))
    sig = jnp.concatenate(sig_parts, axis=0)
    x = jnp.concatenate(x_parts, axis=0)
    for w_ref, b_ref in ((w1_ref, b1_ref), (w2_ref, b2_ref),
                         (w3_ref, b3_ref), (w4_ref, b4_ref)):
        x = jnp.maximum(
            jnp.dot(x, w_ref[...], preferred_element_type=jnp.float32)
            + b_ref[...], 0.0)
    feat = jnp.concatenate([sig, x], axis=1)
    scores = jnp.dot(feat, wd_ref[...], preferred_element_type=jnp.float32) \
        + bd_ref[...]
    s = [scores[k * r:(k + 1) * r] for k in range(NITEM)]
    out_ref[...] = jnp.concatenate(
        [s[0], s[0], s[0], s[0], s[1], s[2], s[3], s[4]], axis=1)


def _dense(u_rows, it_rows3, W1, b1, W2, b2, W3, b3, W4, b4, Wd, bd):
    B = u_rows.shape[0]
    R = 512
    grid = (B // R,)
    full = lambda shape: pl.BlockSpec(shape, lambda i: tuple(0 for _ in shape))
    in_specs = [
        pl.BlockSpec((R, 2 * D), lambda i: (i, 0)),
        pl.BlockSpec((NITEM, R, 2 * D), lambda i: (0, i, 0)),
        full(W1.shape), full((1, b1.shape[0])),
        full(W2.shape), full((1, b2.shape[0])),
        full(W3.shape), full((1, b3.shape[0])),
        full(W4.shape), full((1, b4.shape[0])),
        full(Wd.shape), full((1, 1)),
    ]
    return pl.pallas_call(
        _dense_body,
        grid=grid,
        in_specs=in_specs,
        out_specs=pl.BlockSpec((R, 2 * NNEG), lambda i: (i, 0)),
        out_shape=jax.ShapeDtypeStruct((B, 2 * NNEG), jnp.float32),
    )(u_rows, it_rows3,
      W1, b1.reshape(1, -1), W2, b2.reshape(1, -1),
      W3, b3.reshape(1, -1), W4, b4.reshape(1, -1),
      Wd, bd.reshape(1, 1))


def kernel(user, pos_item, neg_item, mf_user_table, mf_item_table,
           mlp_user_table, mlp_item_table,
           W1, b1, W2, b2, W3, b3, W4, b4, Wd, bd):
    B = user.shape[0]
    nc, ns = _sc_worker_count()
    nw = nc * ns
    user1d = user.astype(jnp.int32)
    items = jnp.concatenate(
        [pos_item.astype(jnp.int32)[None, :], neg_item.astype(jnp.int32).T],
        axis=0)
    items1d = items.reshape(NITEM * B)
    u_comb, i_comb = _pair_concat_t(
        mf_user_table.T, mlp_user_table.T,
        mf_item_table.T, mlp_item_table.T, 8192)
    gk = _make_gather(B, nc, ns)
    u_rows, it_rows = gk(user1d, items1d, u_comb, i_comb)
    it_rows3 = it_rows.reshape(NITEM, B, 2 * D)
    return _dense(u_rows, it_rows3,
                  W1, b1, W2, b2, W3, b3, W4, b4, Wd, bd)


# speedup vs baseline: 1.6983x; 1.0024x over previous
import functools

import jax
import jax.numpy as jnp
from jax import lax
from jax.experimental import pallas as pl
from jax.experimental.pallas import tpu as pltpu
from jax.experimental.pallas import tpu_sc as plsc

D = 64
NNEG = 4
NITEM = NNEG + 1


def _sc_worker_count():
    try:
        info = plsc.get_sparse_core_info()
        return info.num_cores, info.num_subcores
    except Exception:
        return 2, 16


def _pair_body(au_ref, bu_ref, ai_ref, bi_ref, outu_ref, outi_ref):
    outu_ref[...] = jnp.concatenate([au_ref[...].T, bu_ref[...].T], axis=1)
    outi_ref[...] = jnp.concatenate([ai_ref[...].T, bi_ref[...].T], axis=1)


def _pair_concat_t(au_t, bu_t, ai_t, bi_t, cols_per_block):
    n = au_t.shape[1]
    grid = (pl.cdiv(n, cols_per_block),)
    spec = pl.BlockSpec((D, cols_per_block), lambda i: (0, i))
    ospec = pl.BlockSpec((cols_per_block, 2 * D), lambda i: (i, 0))
    oshape = jax.ShapeDtypeStruct((n, 2 * D), jnp.float32)
    return pl.pallas_call(
        _pair_body,
        grid=grid,
        in_specs=[spec, spec, spec, spec],
        out_specs=[ospec, ospec],
        out_shape=[oshape, oshape],
    )(au_t, bu_t, ai_t, bi_t)


@functools.lru_cache(maxsize=None)
def _make_gather(B, nc, ns):
    nw = nc * ns
    bpw = B // nw
    ipw = NITEM * bpw
    nchunk = NITEM
    mesh = plsc.VectorSubcoreMesh(core_axis_name="c", subcore_axis_name="s")

    @functools.partial(
        pl.kernel,
        mesh=mesh,
        out_type=[
            jax.ShapeDtypeStruct((B, 2 * D), jnp.float32),
            jax.ShapeDtypeStruct((NITEM * B, 2 * D), jnp.float32),
        ],
        scratch_types=[
            pltpu.VMEM((bpw,), jnp.int32),
            pltpu.VMEM((ipw,), jnp.int32),
            pltpu.VMEM((bpw, 2 * D), jnp.float32),
            pltpu.VMEM((ipw, 2 * D), jnp.float32),
            pltpu.SemaphoreType.DMA,
        ],
    )
    def gk(user1d, items1d, u_table, i_table,
           out_u, out_i,
           idx_u, idx_it, r_u, r_it, sem):
        wid = lax.axis_index("s") * nc + lax.axis_index("c")
        pltpu.sync_copy(user1d.at[pl.ds(wid * bpw, bpw)], idx_u)
        pltpu.sync_copy(items1d.at[pl.ds(wid * ipw, ipw)], idx_it)
        cps = [pltpu.async_copy(u_table.at[idx_u], r_u, sem)]
        for j in range(nchunk):
            src = idx_it.at[pl.ds(j * bpw, bpw)]
            dst = pl.ds(j * bpw, bpw)
            cps.append(pltpu.async_copy(i_table.at[src], r_it.at[dst], sem))
        for c in cps:
            c.wait()
        pltpu.sync_copy(r_u, out_u.at[pl.ds(wid * bpw, bpw)])
        pltpu.sync_copy(r_it, out_i.at[pl.ds(wid * ipw, ipw)])

    return gk


def _dense_body(u_ref, it_ref,
                w1_ref, b1_ref, w2_ref, b2_ref, w3_ref, b3_ref,
                w4_ref, b4_ref, wd_ref, bd_ref, out_ref):
    r = u_ref.shape[0]
    u = u_ref[...]
    mfu = u[:, :D]
    mlu = u[:, D:]
    sig_parts = []
    x_parts = []
    for k in range(NITEM):
        it = it_ref[k]
        sig_parts.append(jax.nn.sigmoid(mfu * it[:, :D]))
        x_parts.append(jnp.concatenate([mlu, it[:, D:]], axis=1))
    sig = jnp.concatenate(sig_parts, axis=0)
    x = jnp.concatenate(x_parts, axis=0)
    for w_ref, b_ref in ((w1_ref, b1_ref), (w2_ref, b2_ref),
                         (w3_ref, b3_ref), (w4_ref, b4_ref)):
        x = jnp.maximum(
            jnp.dot(x, w_ref[...], preferred_element_type=jnp.float32)
            + b_ref[...], 0.0)
    feat = jnp.concatenate([sig, x], axis=1)
    scores = jnp.dot(feat, wd_ref[...], preferred_element_type=jnp.float32) \
        + bd_ref[...]
    s = [scores[k * r:(k + 1) * r] for k in range(NITEM)]
    out_ref[...] = jnp.concatenate(
        [s[0], s[0], s[0], s[0], s[1], s[2], s[3], s[4]], axis=1)


def _dense(u_rows, it_rows3, W1, b1, W2, b2, W3, b3, W4, b4, Wd, bd):
    B = u_rows.shape[0]
    R = 512
    grid = (B // R,)
    full = lambda shape: pl.BlockSpec(shape, lambda i: tuple(0 for _ in shape))
    in_specs = [
        pl.BlockSpec((R, 2 * D), lambda i: (i, 0)),
        pl.BlockSpec((NITEM, R, 2 * D), lambda i: (0, i, 0)),
        full(W1.shape), full((1, b1.shape[0])),
        full(W2.shape), full((1, b2.shape[0])),
        full(W3.shape), full((1, b3.shape[0])),
        full(W4.shape), full((1, b4.shape[0])),
        full(Wd.shape), full((1, 1)),
    ]
    return pl.pallas_call(
        _dense_body,
        grid=grid,
        in_specs=in_specs,
        out_specs=pl.BlockSpec((R, 2 * NNEG), lambda i: (i, 0)),
        out_shape=jax.ShapeDtypeStruct((B, 2 * NNEG), jnp.float32),
    )(u_rows, it_rows3,
      W1, b1.reshape(1, -1), W2, b2.reshape(1, -1),
      W3, b3.reshape(1, -1), W4, b4.reshape(1, -1),
      Wd, bd.reshape(1, 1))


def kernel(user, pos_item, neg_item, mf_user_table, mf_item_table,
           mlp_user_table, mlp_item_table,
           W1, b1, W2, b2, W3, b3, W4, b4, Wd, bd):
    B = user.shape[0]
    nc, ns = _sc_worker_count()
    nw = nc * ns
    user1d = user.astype(jnp.int32)
    items = jnp.concatenate(
        [pos_item.astype(jnp.int32)[None, :], neg_item.astype(jnp.int32).T],
        axis=0)
    items1d = items.reshape(NITEM * B)
    u_comb, i_comb = _pair_concat_t(
        mf_user_table.T, mlp_user_table.T,
        mf_item_table.T, mlp_item_table.T, 8192)
    gk = _make_gather(B, nc, ns)
    u_rows, it_rows = gk(user1d, items1d, u_comb, i_comb)
    it_rows3 = it_rows.reshape(NITEM, B, 2 * D)
    return _dense(u_rows, it_rows3,
                  W1, b1, W2, b2, W3, b3, W4, b4, Wd, bd)
